# baseline (device time: 99377 ns/iter reference)
import jax
import jax.numpy as jnp
from jax import lax
from jax.experimental import pallas as pl
from jax.experimental.pallas import tpu as pltpu

N_DEV = 16


def kernel(x, w_mat, scale_x, scale_w):
    m_per, k = x.shape
    _, n_per = w_mat.shape

    wb = w_mat.astype(jnp.bfloat16)
    s = (scale_x * scale_w).reshape(1, 1)

    Q = 4
    half = m_per // Q
    NH = Q * N_DEV
    H = NH // 2 - Q // 2

    def body(x_ref, w_ref, s_ref, out_ref, gat_ref, stage_ref, xv_ref,
             res_ref, x_sem, out_sems, cw_send_sems, cw_recv_sems,
             ccw_send_sems, ccw_recv_sems):
        my = lax.axis_index("i")
        left = lax.rem(my + N_DEV - 1, N_DEV)
        right = lax.rem(my + 1, N_DEV)

        xcp = pltpu.make_async_copy(x_ref, xv_ref, x_sem)
        xcp.start()

        barrier_sem = pltpu.get_barrier_semaphore()
        for nbr in (left, right):
            pl.semaphore_signal(
                barrier_sem, inc=1,
                device_id=(nbr,), device_id_type=pl.DeviceIdType.MESH,
            )

        xcp.wait()
        for u in range(Q):
            stage_ref[u] = xv_ref[pl.ds(u * half, half)].astype(
                jnp.float8_e4m3fn
            )

        pl.semaphore_wait(barrier_sem, 2)

        def chunk_slot(c, u):
            return Q * lax.rem(c + N_DEV, N_DEV) + u

        res_state = {"n": 0, "d": [None, None]}

        def compute(s, src=None):
            slot = res_state["n"] % 2
            if res_state["d"][slot] is not None:
                res_state["d"][slot].wait()
            xb = (gat_ref[s] if src is None else src).astype(jnp.bfloat16)
            acc = jnp.dot(xb, w_ref[...], preferred_element_type=jnp.float32)
            y = acc * s_ref[0, 0]
            res_ref[slot] = y * (1.0 / (1.0 + jnp.exp(-y)))
            d = pltpu.make_async_copy(
                res_ref.at[slot],
                out_ref.at[pl.ds(s * half, half)],
                out_sems.at[slot],
            )
            d.start()
            res_state["d"][slot] = d
            res_state["n"] += 1

        S_cw = [chunk_slot(my - j // Q, j % Q) for j in range(H - 2)]
        S_cw += [chunk_slot(my - 7, 0), chunk_slot(my - 7, 1)]
        R_cw = [chunk_slot(my - 1 - j // Q, j % Q) for j in range(H - 2)]
        R_cw += [chunk_slot(my - 8, 0), chunk_slot(my - 8, 1)]
        S_ccw = [chunk_slot(my + j // Q, j % Q) for j in range(H - 2)]
        S_ccw += [chunk_slot(my + 7, 2), chunk_slot(my + 7, 3)]
        R_ccw = [chunk_slot(my + 1 + j // Q, j % Q) for j in range(H - 2)]
        R_ccw += [chunk_slot(my + 8, 2), chunk_slot(my + 8, 3)]

        def mk(dst_dev, slot, send_sem, recv_sem, src=None):
            return pltpu.make_async_remote_copy(
                src_ref=gat_ref.at[slot] if src is None else src,
                dst_ref=gat_ref.at[slot],
                send_sem=send_sem,
                recv_sem=recv_sem,
                device_id=(dst_dev,),
                device_id_type=pl.DeviceIdType.MESH,
            )

        cw_d = [None] * H
        ccw_d = [None] * H

        def cw_issue(j, src=None):
            cw_d[j] = mk(right, S_cw[j], cw_send_sems.at[j],
                         cw_recv_sems.at[j], src)
            cw_d[j].start()

        def ccw_issue(j, src=None):
            ccw_d[j] = mk(left, S_ccw[j], ccw_send_sems.at[j],
                          ccw_recv_sems.at[j], src)
            ccw_d[j].start()

        for u in range(Q):
            cw_issue(u, stage_ref.at[u])
            ccw_issue(u, stage_ref.at[u])
        for u in range(Q):
            compute(Q * my + u, stage_ref[u])

        for j in range(H):
            cw_d[j].wait_recv()
            if j + Q < H:
                cw_issue(j + Q)
            ccw_d[j].wait_recv()
            if j + Q < H - 2:
                ccw_issue(j + Q)
            if j in (H - 4, H - 3):
                ccw_issue(j + 2)
            compute(R_cw[j])
            compute(R_ccw[j])

        for d in cw_d + ccw_d:
            d.wait_send()
        for d in res_state["d"]:
            d.wait()

    return pl.pallas_call(
        body,
        out_shape=jax.ShapeDtypeStruct((N_DEV * m_per, n_per), jnp.float32),
        in_specs=[
            pl.BlockSpec(memory_space=pl.ANY),
            pl.BlockSpec(memory_space=pltpu.VMEM),
            pl.BlockSpec(memory_space=pltpu.SMEM),
        ],
        out_specs=pl.BlockSpec(memory_space=pl.ANY),
        scratch_shapes=[
            pltpu.VMEM((NH, half, k), jnp.float8_e4m3fn),
            pltpu.VMEM((Q, half, k), jnp.float8_e4m3fn),
            pltpu.VMEM((m_per, k), jnp.float32),
            pltpu.VMEM((2, half, n_per), jnp.float32),
            pltpu.SemaphoreType.DMA,
            pltpu.SemaphoreType.DMA((2,)),
            pltpu.SemaphoreType.DMA((H,)),
            pltpu.SemaphoreType.DMA((H,)),
            pltpu.SemaphoreType.DMA((H,)),
            pltpu.SemaphoreType.DMA((H,)),
        ],
        compiler_params=pltpu.CompilerParams(collective_id=0),
    )(x, wb, s)


# device time: 97803 ns/iter; 1.0161x vs baseline; 1.0161x over previous
import jax
import jax.numpy as jnp
from jax import lax
from jax.experimental import pallas as pl
from jax.experimental.pallas import tpu as pltpu

N_DEV = 16


def kernel(x, w_mat, scale_x, scale_w):
    m_per, k = x.shape
    _, n_per = w_mat.shape

    x8 = x.astype(jnp.float8_e4m3fn)
    wb = w_mat.astype(jnp.bfloat16)
    s = (scale_x * scale_w).reshape(1, 1)

    Q = 4
    half = m_per // Q
    NH = Q * N_DEV
    H = NH // 2 - Q // 2

    def body(x_ref, w_ref, s_ref, out_ref, gat_ref,
             cw_send_sems, cw_recv_sems, ccw_send_sems, ccw_recv_sems):
        my = lax.axis_index("i")
        left = lax.rem(my + N_DEV - 1, N_DEV)
        right = lax.rem(my + 1, N_DEV)

        barrier_sem = pltpu.get_barrier_semaphore()
        for nbr in (left, right):
            pl.semaphore_signal(
                barrier_sem, inc=1,
                device_id=(nbr,), device_id_type=pl.DeviceIdType.MESH,
            )
        pl.semaphore_wait(barrier_sem, 2)

        def chunk_slot(c, u):
            return Q * lax.rem(c + N_DEV, N_DEV) + u

        def compute(s, src=None):
            xb = (gat_ref[s] if src is None else src).astype(jnp.bfloat16)
            acc = jnp.dot(xb, w_ref[...], preferred_element_type=jnp.float32)
            y = acc * s_ref[0, 0]
            out_ref[pl.ds(s * half, half), :] = y * (
                1.0 / (1.0 + jnp.exp(-y))
            )

        S_cw = [chunk_slot(my - j // Q, j % Q) for j in range(H - 2)]
        S_cw += [chunk_slot(my - 7, 0), chunk_slot(my - 7, 1)]
        R_cw = [chunk_slot(my - 1 - j // Q, j % Q) for j in range(H - 2)]
        R_cw += [chunk_slot(my - 8, 0), chunk_slot(my - 8, 1)]
        S_ccw = [chunk_slot(my + j // Q, j % Q) for j in range(H - 2)]
        S_ccw += [chunk_slot(my + 7, 2), chunk_slot(my + 7, 3)]
        R_ccw = [chunk_slot(my + 1 + j // Q, j % Q) for j in range(H - 2)]
        R_ccw += [chunk_slot(my + 8, 2), chunk_slot(my + 8, 3)]

        def mk(dst_dev, slot, send_sem, recv_sem, src=None):
            return pltpu.make_async_remote_copy(
                src_ref=gat_ref.at[slot] if src is None else src,
                dst_ref=gat_ref.at[slot],
                send_sem=send_sem,
                recv_sem=recv_sem,
                device_id=(dst_dev,),
                device_id_type=pl.DeviceIdType.MESH,
            )

        cw_d = [None] * H
        ccw_d = [None] * H

        def cw_issue(j, src=None):
            cw_d[j] = mk(right, S_cw[j], cw_send_sems.at[j],
                         cw_recv_sems.at[j], src)
            cw_d[j].start()

        def ccw_issue(j, src=None):
            ccw_d[j] = mk(left, S_ccw[j], ccw_send_sems.at[j],
                          ccw_recv_sems.at[j], src)
            ccw_d[j].start()

        for u in range(Q):
            own = x_ref.at[pl.ds(u * half, half)]
            cw_issue(u, own)
            ccw_issue(u, own)
        for u in range(Q):
            compute(Q * my + u, x_ref[pl.ds(u * half, half)])

        for j in range(H):
            cw_d[j].wait_recv()
            if j + Q < H:
                cw_issue(j + Q)
            ccw_d[j].wait_recv()
            if j + Q < H - 2:
                ccw_issue(j + Q)
            if j in (H - 4, H - 3):
                ccw_issue(j + 2)
            compute(R_cw[j])
            compute(R_ccw[j])

        for d in cw_d + ccw_d:
            d.wait_send()

    return pl.pallas_call(
        body,
        out_shape=jax.ShapeDtypeStruct((N_DEV * m_per, n_per), jnp.float32),
        in_specs=[
            pl.BlockSpec(memory_space=pltpu.VMEM),
            pl.BlockSpec(memory_space=pltpu.VMEM),
            pl.BlockSpec(memory_space=pltpu.SMEM),
        ],
        out_specs=pl.BlockSpec(memory_space=pltpu.VMEM),
        scratch_shapes=[
            pltpu.VMEM((NH, half, k), jnp.float8_e4m3fn),
            pltpu.SemaphoreType.DMA((H,)),
            pltpu.SemaphoreType.DMA((H,)),
            pltpu.SemaphoreType.DMA((H,)),
            pltpu.SemaphoreType.DMA((H,)),
        ],
        compiler_params=pltpu.CompilerParams(collective_id=0),
    )(x8, wb, s)
